# baseline (speedup 1.0000x reference)
import jax
import jax.numpy as jnp
from jax import lax
from jax.experimental import pallas as pl
from jax.experimental.pallas import tpu as pltpu
from jax.experimental.pallas import tpu_sc as plsc

NCHUNK = 4
BLK_B = 4
TB = 8


def kernel(atomic_numbers, embedding_table):
    B, S = atomic_numbers.shape
    V, D = embedding_table.shape
    CB = B // NCHUNK
    dtype = embedding_table.dtype

    mesh = plsc.VectorSubcoreMesh(core_axis_name="c", subcore_axis_name="s")

    @pl.kernel(
        out_type=jax.ShapeDtypeStruct((CB * S, D), dtype),
        mesh=mesh,
        scratch_types=[
            pltpu.VMEM_SHARED((V, D), dtype),
            pltpu.SemaphoreType.DMA,
        ],
    )
    def gather_kernel(table_hbm, idx_hbm, out_hbm, table_spmem, sem):
        @pl.when(lax.axis_index("s") == 0)
        def _():
            pltpu.sync_copy(table_hbm, table_spmem)

        plsc.subcore_barrier()

        def body(i_vmem, o_vmem):
            copies = [
                pltpu.async_copy(
                    table_spmem.at[i_vmem.at[k]],
                    o_vmem.at[pl.ds(k * S, S)],
                    sem,
                )
                for k in range(BLK_B)
            ]
            for c in copies:
                c.wait()

        pltpu.emit_pipeline(
            body,
            grid=(CB // BLK_B,),
            in_specs=[pl.BlockSpec((BLK_B, S), index_map=lambda i: (i, 0))],
            out_specs=[
                pl.BlockSpec((BLK_B * S, D), index_map=lambda i: (i, 0))
            ],
            core_axis_name=("c", "s"),
            dimension_semantics=(pltpu.PARALLEL,),
        )(idx_hbm, out_hbm)

    def make_relayout(c, with_acc):
        def body(*refs):
            ch_ref, o_ref = refs[-2], refs[-1]
            for s in range(TB):
                o_ref[s] = ch_ref[pl.ds(s * S, S), :]

        in_specs = []
        if with_acc:
            in_specs.append(pl.BlockSpec(memory_space=pltpu.MemorySpace.HBM))
        in_specs.append(
            pl.BlockSpec((TB * S, D), index_map=lambda i: (i, 0))
        )
        return pl.pallas_call(
            body,
            grid=(CB // TB,),
            in_specs=in_specs,
            out_specs=pl.BlockSpec(
                (TB, S, D),
                index_map=lambda i, c=c: (c * (CB // TB) + i, 0, 0),
            ),
            out_shape=jax.ShapeDtypeStruct((B, S, D), dtype),
            input_output_aliases={0: 0} if with_acc else {},
        )

    chunks = [
        gather_kernel(embedding_table, atomic_numbers[c * CB:(c + 1) * CB])
        for c in range(NCHUNK)
    ]
    out = make_relayout(0, False)(chunks[0])
    for c in range(1, NCHUNK):
        out = make_relayout(c, True)(out, chunks[c])
    return out


# speedup vs baseline: 1.2414x; 1.2414x over previous
import jax
import jax.numpy as jnp
from jax import lax
from jax.experimental import pallas as pl
from jax.experimental.pallas import tpu as pltpu
from jax.experimental.pallas import tpu_sc as plsc

NCHUNK = 4
BLK_B = 4


def kernel(atomic_numbers, embedding_table):
    B, S = atomic_numbers.shape
    V, D = embedding_table.shape
    CB = B // NCHUNK
    dtype = embedding_table.dtype

    mesh = plsc.VectorSubcoreMesh(core_axis_name="c", subcore_axis_name="s")

    @pl.kernel(
        out_type=jax.ShapeDtypeStruct((CB, S, D), dtype),
        mesh=mesh,
        scratch_types=[
            pltpu.VMEM_SHARED((V, D), dtype),
            pltpu.SemaphoreType.DMA,
        ],
    )
    def gather_kernel(table_hbm, idx_hbm, out_hbm, table_spmem, sem):
        @pl.when(lax.axis_index("s") == 0)
        def _():
            pltpu.sync_copy(table_hbm, table_spmem)

        plsc.subcore_barrier()

        def body(i_vmem, o_vmem):
            copies = [
                pltpu.async_copy(
                    table_spmem.at[i_vmem.at[k]],
                    o_vmem.at[k],
                    sem,
                )
                for k in range(BLK_B)
            ]
            for c in copies:
                c.wait()

        pltpu.emit_pipeline(
            body,
            grid=(CB // BLK_B,),
            in_specs=[pl.BlockSpec((BLK_B, S), index_map=lambda i: (i, 0))],
            out_specs=[
                pl.BlockSpec((BLK_B, S, D), index_map=lambda i: (i, 0, 0))
            ],
            core_axis_name=("c", "s"),
            dimension_semantics=(pltpu.PARALLEL,),
        )(idx_hbm, out_hbm)

    chunks = [
        gather_kernel(embedding_table, atomic_numbers[c * CB:(c + 1) * CB])
        for c in range(NCHUNK)
    ]
    return jnp.concatenate(chunks, axis=0)


# speedup vs baseline: 2.3533x; 1.8956x over previous
import jax
import jax.numpy as jnp
from jax import lax
from jax.experimental import pallas as pl
from jax.experimental.pallas import tpu as pltpu
from jax.experimental.pallas import tpu_sc as plsc

BLK_B = 4


def kernel(atomic_numbers, embedding_table):
    B, S = atomic_numbers.shape
    V, D = embedding_table.shape
    idx = atomic_numbers.astype(jnp.int32)
    dtype = embedding_table.dtype

    mesh = plsc.VectorSubcoreMesh(core_axis_name="c", subcore_axis_name="s")

    @pl.kernel(
        out_type=jax.ShapeDtypeStruct((B, S, D), dtype),
        mesh=mesh,
        scratch_types=[
            pltpu.VMEM_SHARED((V, D), dtype),
            pltpu.SemaphoreType.DMA,
        ],
    )
    def gather_kernel(table_hbm, idx_hbm, out_hbm, table_spmem, sem):
        @pl.when(lax.axis_index("s") == 0)
        def _():
            pltpu.sync_copy(table_hbm, table_spmem)

        plsc.subcore_barrier()

        def body(i_vmem, o_vmem):
            copies = [
                pltpu.async_copy(
                    table_spmem.at[i_vmem.at[k]],
                    o_vmem.at[k],
                    sem,
                )
                for k in range(BLK_B)
            ]
            for c in copies:
                c.wait()

        pltpu.emit_pipeline(
            body,
            grid=(B // BLK_B,),
            in_specs=[pl.BlockSpec((BLK_B, S), index_map=lambda i: (i, 0))],
            out_specs=[
                pl.BlockSpec((BLK_B, S, D), index_map=lambda i: (i, 0, 0))
            ],
            core_axis_name=("c", "s"),
            dimension_semantics=(pltpu.PARALLEL,),
        )(idx_hbm, out_hbm)

    return gather_kernel(embedding_table, idx)
